# baseline (speedup 1.0000x reference)
import jax
import jax.numpy as jnp
from jax import lax
from jax.experimental import pallas as pl
from jax.experimental.pallas import tpu as pltpu
from jax.experimental.pallas import tpu_sc as plsc

_N = 1000000
_B = 16384
_F = 64
_NW = 32
_BPW = _B // _NW
_G = 16
_GROUPS = _BPW // _G
_CHUNK = 128
_NCHUNK = _BPW // _CHUNK


def _user_body(user_hbm, utab_hbm, eut_hbm,
               uidx2_v, urows_v, eut_v, sem):
    nc = 2
    wid = lax.axis_index("s") * nc + lax.axis_index("c")
    base = wid * _BPW

    pltpu.sync_copy(user_hbm.at[pl.ds(wid * _NCHUNK, _NCHUNK)], uidx2_v)

    descs = []
    for k in range(_NCHUNK):
        descs.append(pltpu.async_copy(
            utab_hbm.at[uidx2_v.at[k]],
            urows_v.at[pl.ds(k * _CHUNK, _CHUNK)], sem))
    for d in descs:
        d.wait()

    lane = lax.iota(jnp.int32, 16)

    def group_body(g, carry):
        goff = g * _G
        rows = goff + lane
        for f in range(_F):
            colf = jnp.full((16,), f, jnp.int32)
            eut_v[f, pl.ds(goff, _G)] = plsc.load_gather(urows_v, [rows, colf])
        return carry

    lax.fori_loop(0, _GROUPS, group_body, 0, unroll=False)

    pltpu.sync_copy(eut_v, eut_hbm.at[:, pl.ds(base, _BPW)])


def _item_body(item_hbm, itab_hbm, eut_hbm, wb_hbm, out_hbm,
               iidx_v, eut_v, ibuf_v, wb_v, out_v, sem, sem2):
    nc = 2
    wid = lax.axis_index("s") * nc + lax.axis_index("c")
    base = wid * _BPW

    pltpu.sync_copy(item_hbm.at[pl.ds(base, _BPW)], iidx_v)
    pltpu.sync_copy(wb_hbm, wb_v)
    eut_desc = pltpu.make_async_copy(eut_hbm.at[:, pl.ds(base, _BPW)],
                                     eut_v, sem2)
    eut_desc.start()

    wvecs = [wb_v[pl.ds(c * 16, 16)] for c in range(_F // 16)]
    bvec = wb_v[pl.ds(_F, 16)]
    lane = lax.iota(jnp.int32, 16)

    def fire_group(g):
        buf = lax.rem(g, 2)
        irows = iidx_v[pl.ds(g * _G, _G)]
        for j in range(_G):
            ib = pl.multiple_of(jnp.bitwise_and(irows[j], -8), 8)
            pltpu.async_copy(itab_hbm.at[pl.ds(ib, 8), :],
                             ibuf_v.at[buf, pl.ds(j * 8, 8), :], sem)

    def drain_group(g):
        buf = lax.rem(g, 2)
        for j in range(_G):
            pltpu.make_async_copy(itab_hbm.at[pl.ds(0, 8), :],
                                  ibuf_v.at[buf, pl.ds(j * 8, 8), :], sem).wait()

    def compute_group(g):
        buf = lax.rem(g, 2)
        goff = g * _G
        isub = jnp.bitwise_and(iidx_v[pl.ds(goff, _G)], 7) + lane * 8
        acc = bvec
        for f in range(_F):
            wf = wvecs[f // 16][f % 16]
            colf = jnp.full((16,), f, jnp.int32)
            iv = plsc.load_gather(ibuf_v.at[buf], [isub, colf])
            u = eut_v[f, pl.ds(goff, _G)]
            acc = acc + u * iv * wf
        out_v[pl.ds(goff, _G)] = acc

    fire_group(0)
    eut_desc.wait()

    def group_body(g, carry):
        fire_group(g + 1)
        drain_group(g)
        compute_group(g)
        return carry

    lax.fori_loop(0, _GROUPS - 1, group_body, 0, unroll=False)
    drain_group(_GROUPS - 1)
    compute_group(_GROUPS - 1)

    pltpu.sync_copy(out_v, out_hbm.at[pl.ds(base, _BPW)])


def kernel(user, item, embed_user_w, embed_item_w, predict_w, predict_b):
    wb = jnp.concatenate([predict_w.reshape(_F).astype(jnp.float32),
                          jnp.broadcast_to(predict_b.astype(jnp.float32), (16,))])

    mesh = plsc.VectorSubcoreMesh(core_axis_name="c", subcore_axis_name="s")
    run_user = pl.kernel(
        _user_body,
        out_type=jax.ShapeDtypeStruct((_F, _B), jnp.float32),
        mesh=mesh,
        compiler_params=pltpu.CompilerParams(needs_layout_passes=False,
                                             use_tc_tiling_on_sc=False),
        scratch_types=[
            pltpu.VMEM((_NCHUNK, _CHUNK), jnp.int32),
            pltpu.VMEM((_BPW, _F), jnp.float32),
            pltpu.VMEM((_F, _BPW), jnp.float32),
            pltpu.SemaphoreType.DMA,
        ],
    )
    user2d = user.astype(jnp.int32).reshape(_B // _CHUNK, _CHUNK)
    eut = run_user(user2d, embed_user_w)

    run_item = pl.kernel(
        _item_body,
        out_type=jax.ShapeDtypeStruct((_B,), jnp.float32),
        mesh=mesh,
        compiler_params=pltpu.CompilerParams(needs_layout_passes=False,
                                             use_tc_tiling_on_sc=True),
        scratch_types=[
            pltpu.VMEM((_BPW,), jnp.int32),
            pltpu.VMEM((_F, _BPW), jnp.float32),
            pltpu.VMEM((2, _G * 8, _F), jnp.float32),
            pltpu.VMEM((_F + 16,), jnp.float32),
            pltpu.VMEM((_BPW,), jnp.float32),
            pltpu.SemaphoreType.DMA,
            pltpu.SemaphoreType.DMA,
        ],
    )
    return run_item(item.astype(jnp.int32), embed_item_w, eut, wb)


# speedup vs baseline: 1.3315x; 1.3315x over previous
import jax
import jax.numpy as jnp
from jax import lax
from jax.experimental import pallas as pl
from jax.experimental.pallas import tpu as pltpu
from jax.experimental.pallas import tpu_sc as plsc

_N = 1000000
_B = 16384
_F = 64
_NW = 32
_BPW = _B // _NW
_G = 16
_GROUPS = _BPW // _G


def _clmf_body(user_hbm, item_hbm, utab_hbm, itab_hbm, wb_hbm, out_hbm,
               uidx_v, iidx_v, ubuf_v, ibuf_v, wb_v, out_v, sem):
    nc = 2
    wid = lax.axis_index("s") * nc + lax.axis_index("c")
    base = wid * _BPW

    pltpu.sync_copy(user_hbm.at[pl.ds(base, _BPW)], uidx_v)
    pltpu.sync_copy(item_hbm.at[pl.ds(base, _BPW)], iidx_v)
    pltpu.sync_copy(wb_hbm, wb_v)

    wvecs = [wb_v[pl.ds(c * 16, 16)] for c in range(_F // 16)]
    bvec = wb_v[pl.ds(_F, 16)]
    lane = lax.iota(jnp.int32, 16)

    def fire_group(g):
        buf = lax.rem(g, 2)
        goff = g * _G
        urows = uidx_v[pl.ds(goff, _G)]
        irows = iidx_v[pl.ds(goff, _G)]
        for j in range(_G):
            ub = pl.multiple_of(jnp.bitwise_and(urows[j], -8), 8)
            ib = pl.multiple_of(jnp.bitwise_and(irows[j], -8), 8)
            pltpu.async_copy(utab_hbm.at[pl.ds(ub, 8), :],
                             ubuf_v.at[buf, pl.ds(j * 8, 8), :], sem)
            pltpu.async_copy(itab_hbm.at[pl.ds(ib, 8), :],
                             ibuf_v.at[buf, pl.ds(j * 8, 8), :], sem)

    def drain_group(g):
        buf = lax.rem(g, 2)
        for j in range(_G):
            pltpu.make_async_copy(utab_hbm.at[pl.ds(0, 8), :],
                                  ubuf_v.at[buf, pl.ds(j * 8, 8), :], sem).wait()
            pltpu.make_async_copy(itab_hbm.at[pl.ds(0, 8), :],
                                  ibuf_v.at[buf, pl.ds(j * 8, 8), :], sem).wait()

    def compute_group(g):
        buf = lax.rem(g, 2)
        goff = g * _G
        usub = jnp.bitwise_and(uidx_v[pl.ds(goff, _G)], 7) + lane * 8
        isub = jnp.bitwise_and(iidx_v[pl.ds(goff, _G)], 7) + lane * 8
        acc = bvec
        for f in range(_F):
            wf = wvecs[f // 16][f % 16]
            colf = jnp.full((16,), f, jnp.int32)
            u = plsc.load_gather(ubuf_v.at[buf], [usub, colf])
            iv = plsc.load_gather(ibuf_v.at[buf], [isub, colf])
            acc = acc + u * iv * wf
        out_v[pl.ds(goff, _G)] = acc

    fire_group(0)

    def group_body(g, carry):
        fire_group(g + 1)
        drain_group(g)
        compute_group(g)
        return carry

    lax.fori_loop(0, _GROUPS - 1, group_body, 0, unroll=False)
    drain_group(_GROUPS - 1)
    compute_group(_GROUPS - 1)

    pltpu.sync_copy(out_v, out_hbm.at[pl.ds(base, _BPW)])


def kernel(user, item, embed_user_w, embed_item_w, predict_w, predict_b):
    wb = jnp.concatenate([predict_w.reshape(_F).astype(jnp.float32),
                          jnp.broadcast_to(predict_b.astype(jnp.float32), (16,))])

    mesh = plsc.VectorSubcoreMesh(core_axis_name="c", subcore_axis_name="s")
    run = pl.kernel(
        _clmf_body,
        out_type=jax.ShapeDtypeStruct((_B,), jnp.float32),
        mesh=mesh,
        compiler_params=pltpu.CompilerParams(needs_layout_passes=False,
                                             use_tc_tiling_on_sc=True),
        scratch_types=[
            pltpu.VMEM((_BPW,), jnp.int32),
            pltpu.VMEM((_BPW,), jnp.int32),
            pltpu.VMEM((2, _G * 8, _F), jnp.float32),
            pltpu.VMEM((2, _G * 8, _F), jnp.float32),
            pltpu.VMEM((_F + 16,), jnp.float32),
            pltpu.VMEM((_BPW,), jnp.float32),
            pltpu.SemaphoreType.DMA,
        ],
    )
    return run(user.astype(jnp.int32), item.astype(jnp.int32),
               embed_user_w, embed_item_w, wb)


# speedup vs baseline: 1.5177x; 1.1399x over previous
import jax
import jax.numpy as jnp
from jax import lax
from jax.experimental import pallas as pl
from jax.experimental.pallas import tpu as pltpu
from jax.experimental.pallas import tpu_sc as plsc

_N = 1000000
_B = 16384
_F = 64
_NW = 32
_BPW = _B // _NW
_G = 16
_GROUPS = _BPW // _G
_NBLK = (_N + 127) // 128
_BPR = (_NBLK + _NW - 1) // _NW
_NCH = _B // 16
_BMCH = 16
_RING = 8


def _gather_body(user_hbm, item_hbm, utab_hbm, itab_hbm, euf_hbm, eif_hbm,
                 allidx_v, cand_v, mb_v, mc_v, bitmap_v, hb_v,
                 blk_v, stage_v, sem, semw):
    nc = 2
    wid = lax.axis_index("s") * nc + lax.axis_index("c")
    lane = lax.iota(jnp.int32, 16)
    lo = wid * _BPR
    hi = jnp.minimum(lo + _BPR, _NBLK)
    nloc = hi - lo
    zeros16 = jnp.zeros((16,), jnp.int32)
    ones16 = jnp.full((16,), 1, jnp.int32)

    now = zeros16

    for idx_hbm, tab_hbm, out_hbm in ((user_hbm, utab_hbm, euf_hbm),
                                      (item_hbm, itab_hbm, eif_hbm)):
        pltpu.sync_copy(idx_hbm, allidx_v)
        for t in range(_BMCH):
            bitmap_v[pl.ds(t * 16, 16)] = zeros16

        def cand_body(k, base):
            idxv = allidx_v[pl.ds(k * 16, 16)]
            bvals = k * 16 + lane
            bid = lax.shift_right_logical(idxv, 7)
            m = jnp.logical_and(bid >= lo, bid < hi)
            mi = m.astype(jnp.int32)
            pos = base + plsc.cumsum(mi) - 1
            plsc.store_scatter(cand_v, [pos], bvals, mask=m)
            plsc.store_scatter(bitmap_v, [bid - lo], ones16, mask=m)
            return base + plsc.all_reduce_population_count(m)

        base = lax.fori_loop(0, _NCH, cand_body, zeros16, unroll=False)
        ncand = base[0]

        def hb_body(t, hbase):
            bm = bitmap_v[pl.ds(t * 16, 16)]
            loc = t * 16 + lane
            m = jnp.logical_and(bm > 0, loc < nloc)
            mi = m.astype(jnp.int32)
            pos = hbase + plsc.cumsum(mi) - 1
            plsc.store_scatter(hb_v, [pos], loc + lo, mask=m)
            return hbase + plsc.all_reduce_population_count(m)

        hbase = lax.fori_loop(0, _BMCH, hb_body, zeros16, unroll=False)
        nhb = hbase[0]
        ncchunk = lax.div(ncand + 15, 16)

        def fetch(t, buf):
            tv = jnp.full((16,), t, jnp.int32)
            blk = plsc.load_gather(hb_v, [tv])[0]
            off = pl.multiple_of(blk * 128, 128)
            pltpu.async_copy(tab_hbm.at[:, pl.ds(off, 128)],
                             blk_v.at[buf], sem)

        def drain(buf):
            pltpu.make_async_copy(tab_hbm.at[:, pl.ds(0, 128)],
                                  blk_v.at[buf], sem).wait()

        def process(t, buf, nw):
            tv = jnp.full((16,), t, jnp.int32)
            blkv = plsc.load_gather(hb_v, [tv])

            def chunk_body(c, nw_c):
                valid = c * 16 + lane < ncand
                cb = plsc.load_gather(cand_v, [c * 16 + lane], mask=valid)
                civ = plsc.load_gather(allidx_v, [cb], mask=valid)
                m = jnp.logical_and(valid,
                                    lax.shift_right_logical(civ, 7) == blkv)
                mi = m.astype(jnp.int32)
                pos = plsc.cumsum(mi) - 1
                plsc.store_scatter(mb_v, [pos], cb, mask=m)
                plsc.store_scatter(mc_v, [pos], jnp.bitwise_and(civ, 127),
                                   mask=m)
                cnt = plsc.all_reduce_population_count(m)

                def match_body(j, carry2):
                    ordv = nw_c + j
                    slot = lax.rem(ordv[0], _RING)

                    @pl.when(ordv[0] >= _RING)
                    def _():
                        pltpu.make_async_copy(
                            euf_hbm.at[pl.ds(0, _F)],
                            stage_v.at[0], semw).wait()

                    jv = jnp.full((16,), j, jnp.int32)
                    bsc = plsc.load_gather(mb_v, [jv])[0]
                    cmod = plsc.load_gather(mc_v, [jv])
                    for k in range(_F // 16):
                        stage_v[slot, pl.ds(k * 16, 16)] = plsc.load_gather(
                            blk_v.at[buf], [k * 16 + lane, cmod])
                    pltpu.async_copy(stage_v.at[slot],
                                     out_hbm.at[pl.ds(bsc * _F, _F)], semw)
                    return carry2

                lax.fori_loop(0, cnt[0], match_body, 0, unroll=False)
                return nw_c + cnt

            return lax.fori_loop(0, ncchunk, chunk_body, nw, unroll=False)

        @pl.when(nhb > 0)
        def _():
            fetch(0, 0)

        def blk_body(t, nw):
            buf = lax.rem(t, 2)

            @pl.when(t + 1 < nhb)
            def _():
                fetch(t + 1, 1 - buf)

            drain(buf)
            return process(t, buf, nw)

        now = lax.fori_loop(0, nhb, blk_body, now, unroll=False)

    def drainw_body(j, carry):
        pltpu.make_async_copy(euf_hbm.at[pl.ds(0, _F)],
                              stage_v.at[0], semw).wait()
        return carry

    lax.fori_loop(0, jnp.minimum(now[0], _RING), drainw_body, 0,
                  unroll=False)


def _combine_body(euf_hbm, eif_hbm, wb_hbm, out_hbm,
                  eu_v, ei_v, wb_v, out_v, sem):
    nc = 2
    wid = lax.axis_index("s") * nc + lax.axis_index("c")
    base = wid * _BPW

    pltpu.sync_copy(euf_hbm.at[pl.ds(base * _F, _BPW * _F)], eu_v)
    pltpu.sync_copy(eif_hbm.at[pl.ds(base * _F, _BPW * _F)], ei_v)
    pltpu.sync_copy(wb_hbm, wb_v)

    wvecs = [wb_v[pl.ds(c * 16, 16)] for c in range(_F // 16)]
    bvec = wb_v[pl.ds(_F, 16)]
    lane = lax.iota(jnp.int32, 16)

    def group_body(g, carry):
        goff = g * _G
        rows = goff + lane
        acc = bvec
        for f in range(_F):
            wf = wvecs[f // 16][f % 16]
            flat = rows * _F + f
            u = plsc.load_gather(eu_v, [flat])
            iv = plsc.load_gather(ei_v, [flat])
            acc = acc + u * iv * wf
        out_v[pl.ds(goff, _G)] = acc
        return carry

    lax.fori_loop(0, _GROUPS, group_body, 0, unroll=False)
    pltpu.sync_copy(out_v, out_hbm.at[pl.ds(base, _BPW)])


def kernel(user, item, embed_user_w, embed_item_w, predict_w, predict_b):
    wb = jnp.concatenate([predict_w.reshape(_F).astype(jnp.float32),
                          jnp.broadcast_to(predict_b.astype(jnp.float32), (16,))])
    mesh = plsc.VectorSubcoreMesh(core_axis_name="c", subcore_axis_name="s")

    run_gather = pl.kernel(
        _gather_body,
        out_type=(jax.ShapeDtypeStruct((_B * _F,), jnp.float32),
                  jax.ShapeDtypeStruct((_B * _F,), jnp.float32)),
        mesh=mesh,
        compiler_params=pltpu.CompilerParams(needs_layout_passes=False,
                                             use_tc_tiling_on_sc=True),
        scratch_types=[
            pltpu.VMEM((_B,), jnp.int32),
            pltpu.VMEM((_B,), jnp.int32),
            pltpu.VMEM((16,), jnp.int32),
            pltpu.VMEM((16,), jnp.int32),
            pltpu.VMEM((_BMCH * 16,), jnp.int32),
            pltpu.VMEM((_BMCH * 16,), jnp.int32),
            pltpu.VMEM((2, _F, 128), jnp.float32),
            pltpu.VMEM((_RING, _F), jnp.float32),
            pltpu.SemaphoreType.DMA,
            pltpu.SemaphoreType.DMA,
        ],
    )
    euf, eif = run_gather(user.astype(jnp.int32), item.astype(jnp.int32),
                          embed_user_w.T, embed_item_w.T)

    run_combine = pl.kernel(
        _combine_body,
        out_type=jax.ShapeDtypeStruct((_B,), jnp.float32),
        mesh=mesh,
        compiler_params=pltpu.CompilerParams(needs_layout_passes=False,
                                             use_tc_tiling_on_sc=False),
        scratch_types=[
            pltpu.VMEM((_BPW * _F,), jnp.float32),
            pltpu.VMEM((_BPW * _F,), jnp.float32),
            pltpu.VMEM((_F + 16,), jnp.float32),
            pltpu.VMEM((_BPW,), jnp.float32),
            pltpu.SemaphoreType.DMA,
        ],
    )
    return run_combine(euf, eif, wb)
